# baseline (speedup 1.0000x reference)
import jax
import jax.numpy as jnp
from jax import lax
from jax.experimental import pallas as pl
from jax.experimental.pallas import tpu as pltpu
from jax.experimental.pallas import tpu_sc as plsc

B = 16384
POS = 20
NEG = 100
CTX = POS + NEG
CTXP = 128
D = 64
DP = 128
NC = 2
NS = 16
NW = NC * NS
PER_W = B // NW
G = 8
NG = PER_W // G
NCHUNK = 8


def _sc_body(out_embed, u_pre, ctx_idx, dots_out,
             u_rows, ctx_idx_v, ctx_rows, dots_v,
             gsem0, gsem1, isem0, isem1, dsem0, dsem1):
    wid = lax.axis_index("s") * NC + lax.axis_index("c")
    lane = lax.broadcasted_iota(jnp.int32, (16,), 0)
    gsem = (gsem0, gsem1)
    isem = (isem0, isem1)
    dsem = (dsem0, dsem1)

    def base_of(g):
        return wid * PER_W + g * G

    def gather_descs(p, g):
        descs = [pltpu.make_async_copy(
            u_pre.at[pl.ds(base_of(g), G), :], u_rows.at[p], gsem[p])]
        for e in range(G):
            descs.append(pltpu.make_async_copy(
                out_embed.at[ctx_idx_v.at[p, e, pl.ds(0, 64)]],
                ctx_rows.at[p, e, pl.ds(0, 64)], gsem[p]))
            descs.append(pltpu.make_async_copy(
                out_embed.at[ctx_idx_v.at[p, e, pl.ds(64, CTX - 64)]],
                ctx_rows.at[p, e, pl.ds(64, CTX - 64)], gsem[p]))
        return descs

    def idx_desc(p, g):
        return pltpu.make_async_copy(
            ctx_idx.at[pl.ds(base_of(g), G), :], ctx_idx_v.at[p], isem[p])

    def dots_desc(p, g):
        return pltpu.make_async_copy(
            dots_v.at[p], dots_out.at[pl.ds(base_of(g), G), :], dsem[p])

    def compute(p, g):
        for e in range(G):
            u0 = u_rows[p, e, pl.ds(0, 16)]
            u1 = u_rows[p, e, pl.ds(16, 16)]
            u2 = u_rows[p, e, pl.ds(32, 16)]
            u3 = u_rows[p, e, pl.ds(48, 16)]

            def chunk(k, _, e=e, u0=u0, u1=u1, u2=u2, u3=u3):
                off = jnp.minimum(k * 16, CTX - 16)
                dots16 = jnp.zeros((16,), jnp.float32)
                for c in range(16):
                    cc = off + c
                    acc = (u0 * ctx_rows[p, e, cc, pl.ds(0, 16)]
                           + u1 * ctx_rows[p, e, cc, pl.ds(16, 16)]
                           + u2 * ctx_rows[p, e, cc, pl.ds(32, 16)]
                           + u3 * ctx_rows[p, e, cc, pl.ds(48, 16)])
                    dots16 = jnp.where(lane == c, jnp.sum(acc), dots16)
                dots_v[p, e, pl.ds(off, 16)] = dots16
                return _

            lax.fori_loop(0, NCHUNK, chunk, None)

    pltpu.sync_copy(ctx_idx.at[pl.ds(base_of(0), G), :], ctx_idx_v.at[0])
    for d in gather_descs(0, 0):
        d.start()
    pltpu.sync_copy(ctx_idx.at[pl.ds(base_of(1), G), :], ctx_idx_v.at[1])

    def step(h, _):
        for b in range(2):
            g = 2 * h + b
            q = 1 - b
            if b == 0:
                @pl.when(h >= 1)
                def _wait_idx0():
                    idx_desc(q, g + 1).wait()
                for d in gather_descs(q, g + 1):
                    d.start()
            else:
                @pl.when(h < NG // 2 - 1)
                def _fire():
                    idx_desc(q, g + 1).wait()
                    for d in gather_descs(q, g + 1):
                        d.start()
            for d in gather_descs(b, g):
                d.wait()
            @pl.when(h < NG // 2 - 1)
            def _stage():
                idx_desc(b, g + 2).start()
            @pl.when(h >= 1)
            def _wait_dots():
                dots_desc(b, g - 2).wait()
            compute(b, g)
            dots_desc(b, g).start()
        return _

    lax.fori_loop(0, NG // 2, step, None)

    dots_desc(0, NG - 2).wait()
    dots_desc(1, NG - 1).wait()


def _tc_logsig_body(dots_ref, out_ref):
    x = dots_ref[...]
    lp = jax.nn.log_sigmoid(x[:, :POS]).sum(axis=1)
    ln = jax.nn.log_sigmoid(-x[:, POS:CTX]).sum(axis=1)
    out_ref[...] = -(lp + ln)


@jax.jit
def kernel(input_labels, pos_labels, neg_labels, in_embed, out_embed):
    u_pre = jnp.pad(jnp.take(in_embed, input_labels, axis=0),
                    ((0, 0), (0, DP - D)))
    ctx_idx = jnp.concatenate(
        [pos_labels.astype(jnp.int32), neg_labels.astype(jnp.int32),
         jnp.zeros((B, CTXP - CTX), jnp.int32)], axis=1)

    mesh = plsc.VectorSubcoreMesh(core_axis_name="c", subcore_axis_name="s")
    dots = pl.kernel(
        _sc_body,
        out_type=jax.ShapeDtypeStruct((B, CTXP), jnp.float32),
        mesh=mesh,
        compiler_params=pltpu.CompilerParams(
            needs_layout_passes=False, use_tc_tiling_on_sc=False),
        scratch_types=[
            pltpu.VMEM((2, G, DP), jnp.float32),
            pltpu.VMEM((2, G, CTXP), jnp.int32),
            pltpu.VMEM((2, G, CTX, D), jnp.float32),
            pltpu.VMEM((2, G, CTXP), jnp.float32),
            pltpu.SemaphoreType.DMA,
            pltpu.SemaphoreType.DMA,
            pltpu.SemaphoreType.DMA,
            pltpu.SemaphoreType.DMA,
            pltpu.SemaphoreType.DMA,
            pltpu.SemaphoreType.DMA,
        ],
    )(out_embed, u_pre, ctx_idx)

    BB = 2048
    loss = pl.pallas_call(
        _tc_logsig_body,
        grid=(B // BB,),
        in_specs=[pl.BlockSpec((BB, CTXP), lambda i: (i, 0))],
        out_specs=pl.BlockSpec((BB,), lambda i: (i,)),
        out_shape=jax.ShapeDtypeStruct((B,), jnp.float32),
    )(dots)
    return loss


# speedup vs baseline: 1.0085x; 1.0085x over previous
import jax
import jax.numpy as jnp
from jax import lax
from jax.experimental import pallas as pl
from jax.experimental.pallas import tpu as pltpu
from jax.experimental.pallas import tpu_sc as plsc

B = 16384
POS = 20
NEG = 100
CTX = POS + NEG
CTXP = 128
D = 64
DP = 128
NC = 2
NS = 16
NW = NC * NS
PER_W = B // NW
G = 8
NG = PER_W // G
NCHUNK = 8


def _sc_body(out_embed, u_pre, ctx_idx, dots_out,
             u_rows, ctx_idx_v, ctx_rows, dots_v,
             gsem0, gsem1, isem0, isem1, dsem0, dsem1):
    wid = lax.axis_index("s") * NC + lax.axis_index("c")
    lane = lax.broadcasted_iota(jnp.int32, (16,), 0)
    gsem = (gsem0, gsem1)
    isem = (isem0, isem1)
    dsem = (dsem0, dsem1)

    def base_of(g):
        return wid * PER_W + g * G

    def gather_descs(p, g):
        descs = [pltpu.make_async_copy(
            u_pre.at[pl.ds(base_of(g), G), :], u_rows.at[p], gsem[p])]
        for e in range(G):
            descs.append(pltpu.make_async_copy(
                out_embed.at[ctx_idx_v.at[p, e, pl.ds(0, CTX)]],
                ctx_rows.at[p, e], gsem[p]))
        return descs

    def idx_desc(p, g):
        return pltpu.make_async_copy(
            ctx_idx.at[pl.ds(base_of(g), G), :], ctx_idx_v.at[p], isem[p])

    def dots_desc(p, g):
        return pltpu.make_async_copy(
            dots_v.at[p], dots_out.at[pl.ds(base_of(g), G), :], dsem[p])

    def compute(p, g):
        for e in range(G):
            u0 = u_rows[p, e, pl.ds(0, 16)]
            u1 = u_rows[p, e, pl.ds(16, 16)]
            u2 = u_rows[p, e, pl.ds(32, 16)]
            u3 = u_rows[p, e, pl.ds(48, 16)]

            def chunk(k, _, e=e, u0=u0, u1=u1, u2=u2, u3=u3):
                off = jnp.minimum(k * 16, CTX - 16)
                dots16 = jnp.zeros((16,), jnp.float32)
                for c in range(16):
                    cc = off + c
                    acc = (u0 * ctx_rows[p, e, cc, pl.ds(0, 16)]
                           + u1 * ctx_rows[p, e, cc, pl.ds(16, 16)]
                           + u2 * ctx_rows[p, e, cc, pl.ds(32, 16)]
                           + u3 * ctx_rows[p, e, cc, pl.ds(48, 16)])
                    dots16 = jnp.where(lane == c, jnp.sum(acc), dots16)
                dots_v[p, e, pl.ds(off, 16)] = dots16
                return _

            lax.fori_loop(0, NCHUNK, chunk, None)

    pltpu.sync_copy(ctx_idx.at[pl.ds(base_of(0), G), :], ctx_idx_v.at[0])
    for d in gather_descs(0, 0):
        d.start()
    pltpu.sync_copy(ctx_idx.at[pl.ds(base_of(1), G), :], ctx_idx_v.at[1])

    def step(h, _):
        for b in range(2):
            g = 2 * h + b
            q = 1 - b
            if b == 0:
                @pl.when(h >= 1)
                def _wait_idx0():
                    idx_desc(q, g + 1).wait()
                for d in gather_descs(q, g + 1):
                    d.start()
            else:
                @pl.when(h < NG // 2 - 1)
                def _fire():
                    idx_desc(q, g + 1).wait()
                    for d in gather_descs(q, g + 1):
                        d.start()
            for d in gather_descs(b, g):
                d.wait()
            @pl.when(h < NG // 2 - 1)
            def _stage():
                idx_desc(b, g + 2).start()
            @pl.when(h >= 1)
            def _wait_dots():
                dots_desc(b, g - 2).wait()
            compute(b, g)
            dots_desc(b, g).start()
        return _

    lax.fori_loop(0, NG // 2, step, None)

    dots_desc(0, NG - 2).wait()
    dots_desc(1, NG - 1).wait()


def _tc_logsig_body(dots_ref, out_ref):
    x = dots_ref[...]
    lp = jax.nn.log_sigmoid(x[:, :POS]).sum(axis=1)
    ln = jax.nn.log_sigmoid(-x[:, POS:CTX]).sum(axis=1)
    out_ref[...] = -(lp + ln)


@jax.jit
def kernel(input_labels, pos_labels, neg_labels, in_embed, out_embed):
    u_pre = jnp.pad(jnp.take(in_embed, input_labels, axis=0),
                    ((0, 0), (0, DP - D)))
    ctx_idx = jnp.concatenate(
        [pos_labels.astype(jnp.int32), neg_labels.astype(jnp.int32),
         jnp.zeros((B, CTXP - CTX), jnp.int32)], axis=1)

    mesh = plsc.VectorSubcoreMesh(core_axis_name="c", subcore_axis_name="s")
    dots = pl.kernel(
        _sc_body,
        out_type=jax.ShapeDtypeStruct((B, CTXP), jnp.float32),
        mesh=mesh,
        compiler_params=pltpu.CompilerParams(
            needs_layout_passes=False, use_tc_tiling_on_sc=False),
        scratch_types=[
            pltpu.VMEM((2, G, DP), jnp.float32),
            pltpu.VMEM((2, G, CTXP), jnp.int32),
            pltpu.VMEM((2, G, CTX, D), jnp.float32),
            pltpu.VMEM((2, G, CTXP), jnp.float32),
            pltpu.SemaphoreType.DMA,
            pltpu.SemaphoreType.DMA,
            pltpu.SemaphoreType.DMA,
            pltpu.SemaphoreType.DMA,
            pltpu.SemaphoreType.DMA,
            pltpu.SemaphoreType.DMA,
        ],
    )(out_embed, u_pre, ctx_idx)

    BB = 2048
    loss = pl.pallas_call(
        _tc_logsig_body,
        grid=(B // BB,),
        in_specs=[pl.BlockSpec((BB, CTXP), lambda i: (i, 0))],
        out_specs=pl.BlockSpec((BB,), lambda i: (i,)),
        out_shape=jax.ShapeDtypeStruct((B,), jnp.float32),
    )(dots)
    return loss
